# Initial kernel scaffold; baseline (speedup 1.0000x reference)
import jax
import jax.numpy as jnp
from jax.experimental import pallas as pl


def kernel(x, log_tau):
    raise NotImplementedError("write your pallas kernel here")


# speedup vs baseline: 1.1933x; 1.1933x over previous
import jax
import jax.numpy as jnp
from jax.experimental import pallas as pl

_B = 16384
_N = 1000
_BR = 512


def _body(x_ref, lt_ref, o_ref):
    xv = x_ref[...]
    m = jnp.max(xv, axis=1, keepdims=True)
    cols = jax.lax.broadcasted_iota(jnp.int32, xv.shape, 1)
    idx = jnp.min(jnp.where(xv == m, cols, _N), axis=1)
    tab = jnp.exp(lt_ref[...])
    onehot = cols == idx[:, None]
    tau = jnp.sum(jnp.where(onehot, tab, 0.0), axis=1)
    o_ref[...] = tau[:, None]


def kernel(x, log_tau):
    lt2 = log_tau.reshape(1, _N)
    out = pl.pallas_call(
        _body,
        grid=(_B // _BR,),
        in_specs=[
            pl.BlockSpec((_BR, _N), lambda i: (i, 0)),
            pl.BlockSpec((1, _N), lambda i: (0, 0)),
        ],
        out_specs=pl.BlockSpec((_BR, 1), lambda i: (i, 0)),
        out_shape=jax.ShapeDtypeStruct((_B, 1), jnp.float32),
    )(x, lt2)
    return out


# speedup vs baseline: 1.3105x; 1.0982x over previous
import jax
import jax.numpy as jnp
from jax.experimental import pallas as pl

_B = 16384
_N = 1000
_BR = 1024


def _body(x_ref, lt_ref, o_ref):
    xv = x_ref[...]
    m = jnp.max(xv, axis=1, keepdims=True)
    cols = jax.lax.broadcasted_iota(jnp.int32, xv.shape, 1)
    idx = jnp.min(jnp.where(xv == m, cols, _N), axis=1)
    tab = jnp.exp(lt_ref[...])
    onehot = cols == idx[:, None]
    tau = jnp.sum(jnp.where(onehot, tab, 0.0), axis=1)
    o_ref[...] = tau[:, None]


def kernel(x, log_tau):
    lt2 = log_tau.reshape(1, _N)
    out = pl.pallas_call(
        _body,
        grid=(_B // _BR,),
        in_specs=[
            pl.BlockSpec((_BR, _N), lambda i: (i, 0)),
            pl.BlockSpec((1, _N), lambda i: (0, 0)),
        ],
        out_specs=pl.BlockSpec((_BR, 1), lambda i: (i, 0)),
        out_shape=jax.ShapeDtypeStruct((_B, 1), jnp.float32),
    )(x, lt2)
    return out


# speedup vs baseline: 1.3572x; 1.0357x over previous
import jax
import jax.numpy as jnp
from jax.experimental import pallas as pl

_B = 16384
_N = 1000
_BR = 2048


def _body(x_ref, lt_ref, o_ref):
    xv = x_ref[...]
    m = jnp.max(xv, axis=1, keepdims=True)
    cols = jax.lax.broadcasted_iota(jnp.int32, xv.shape, 1)
    idx = jnp.min(jnp.where(xv == m, cols, _N), axis=1)
    tab = jnp.exp(lt_ref[...])
    onehot = cols == idx[:, None]
    tau = jnp.sum(jnp.where(onehot, tab, 0.0), axis=1)
    o_ref[...] = tau[:, None]


def kernel(x, log_tau):
    lt2 = log_tau.reshape(1, _N)
    out = pl.pallas_call(
        _body,
        grid=(_B // _BR,),
        in_specs=[
            pl.BlockSpec((_BR, _N), lambda i: (i, 0)),
            pl.BlockSpec((1, _N), lambda i: (0, 0)),
        ],
        out_specs=pl.BlockSpec((_BR, 1), lambda i: (i, 0)),
        out_shape=jax.ShapeDtypeStruct((_B, 1), jnp.float32),
    )(x, lt2)
    return out
